# baseline (speedup 1.0000x reference)
import functools

import jax
import jax.numpy as jnp
from jax import lax
from jax.experimental import pallas as pl
from jax.experimental.pallas import tpu as pltpu
from jax.experimental.pallas import tpu_sc as plsc

NUM_PROMPTS = 1000
SEQ_LEN = 77
TEXT_DIM = 4096
BATCH = 256

NW = 32
ROWS_PER_TILE = BATCH // NW
NBLK = 10
NALIGNED = NBLK - 1
TAIL = SEQ_LEN - 8 * NALIGNED
NBUF = 3
UNITS = [(r, c) for r in range(ROWS_PER_TILE) for c in range(NBLK)]
NSLOT = ROWS_PER_TILE * NALIGNED * 8

_mesh = plsc.VectorSubcoreMesh(core_axis_name="c", subcore_axis_name="s")


@functools.partial(
    pl.kernel,
    mesh=_mesh,
    out_type=jax.ShapeDtypeStruct((BATCH, SEQ_LEN, TEXT_DIM), jnp.float32),
    compiler_params=pltpu.CompilerParams(needs_layout_passes=False),
    scratch_types=[
        pltpu.VMEM((BATCH + 16,), jnp.int32),
        pltpu.VMEM((NSLOT,), jnp.int32),
        pltpu.VMEM((1, 8, TEXT_DIM), jnp.float32),
        pltpu.VMEM((1, 8, TEXT_DIM), jnp.float32),
        pltpu.VMEM((1, 8, TEXT_DIM), jnp.float32),
        pltpu.SemaphoreType.DMA,
        pltpu.SemaphoreType.DMA,
        pltpu.SemaphoreType.DMA,
        pltpu.SemaphoreType.DMA,
        pltpu.SemaphoreType.DMA,
    ],
)
def _sc_gather(table, idx_hbm, out, idx_v, slots, b0, b1, b2,
               gs, gt, s0, s1, s2):
    wid = lax.axis_index("s") * 2 + lax.axis_index("c")
    rbase = wid * ROWS_PER_TILE

    pltpu.sync_copy(idx_hbm, idx_v.at[pl.ds(0, BATCH)])
    ids = idx_v[pl.ds(rbase, 16)]

    for k in range((NSLOT // 8 + 15) // 16):
        u = k * 16 + lax.iota(jnp.int32, 16)
        u = jnp.minimum(u, NSLOT // 8 - 1)
        r = lax.shift_right_logical(u * 7282, 16)
        val = plsc.load_gather(idx_v, [rbase + r])
        plsc.store_scatter(slots, [u * 8], val)

    bufs = [(b0, s0), (b1, s1), (b2, s2)]

    def blk(r, c):
        sl = 8 if c < NALIGNED else TAIL
        return r, c * 8, sl

    def src(i):
        r, off, sl = blk(*UNITS[i])
        if sl == 8:
            slot = (r * NALIGNED + UNITS[i][1]) * 8
            return table.at[slots.at[pl.ds(slot, 1)], pl.ds(off, sl), :]
        return table.at[pl.ds(ids[r], 1), pl.ds(off, sl), :]

    def gsem(i):
        return gs if UNITS[i][1] < NALIGNED else gt

    def buf_view(i):
        _, _, sl = blk(*UNITS[i])
        buf, sem = bufs[i % NBUF]
        return (buf if sl == 8 else buf.at[:, pl.ds(0, sl), :]), sem

    def dst(i):
        r, off, sl = blk(*UNITS[i])
        return out.at[pl.ds(rbase + r, 1), pl.ds(off, sl), :]

    def start_gather(i):
        bv, _ = buf_view(i)
        if i >= NBUF:
            pv, sem = buf_view(i - NBUF)
            pltpu.make_async_copy(pv, dst(i - NBUF), sem).wait()
        pltpu.async_copy(src(i), bv, gsem(i))

    n = len(UNITS)
    start_gather(0)
    for i in range(n):
        if i + 1 < n:
            start_gather(i + 1)
        bv, sem = buf_view(i)
        pltpu.make_async_copy(src(i), bv, gsem(i)).wait()
        pltpu.async_copy(bv, dst(i), sem)

    for i in range(n - NBUF, n):
        pv, sem = buf_view(i)
        pltpu.make_async_copy(pv, dst(i), sem).wait()


def kernel(prompt_idx, embeddings):
    return _sc_gather(embeddings, prompt_idx.astype(jnp.int32))
